# baseline (speedup 1.0000x reference)
import functools

import jax
import jax.numpy as jnp
from jax import lax
from jax.experimental import pallas as pl
from jax.experimental.pallas import tpu as pltpu
from jax.experimental.pallas import tpu_sc as plsc

BB, NN, SS, CC = 4, 1024, 64, 64
HB = 2
NH = BB // HB
CIN = 2 * CC + 3
TQ = 1024
EPS = 1e-3
BIG = 3.0e38

_HIGH = lax.Precision.HIGHEST


def _k1_body(xyz1_ref, x1t_ref, x2t_ref, f1t_ref, f2t_ref, w1t_ref, b1_ref,
             ind_ref, tab_ref, q_ref, stat_ref, dscr, tabf_ref, tabsq_ref):
    b = pl.program_id(0)
    t = pl.program_id(1)

    @pl.when(t == 0)
    def _tables():
        x2t = x2t_ref[...].reshape(NN, 3)
        f2t = f2t_ref[...].reshape(NN, CC)
        acc = jnp.dot(f2t, w1t_ref[3:3 + CC, :], precision=_HIGH,
                      preferred_element_type=jnp.float32)
        for c in range(3):
            acc = acc + x2t[:, c:c + 1] * w1t_ref[c:c + 1, :]
        tab_ref[...] = jnp.concatenate(
            [acc, jnp.zeros((NN, CC), jnp.float32)], axis=1
        ).reshape(1, NN, 2 * CC)
        tabf_ref[...] = acc
        tabsq_ref[...] = acc * acc

        x1t = x1t_ref[...].reshape(NN, 3)
        f1t = f1t_ref[...].reshape(NN, CC)
        q = jnp.dot(f1t, w1t_ref[3 + CC:CIN, :], precision=_HIGH,
                    preferred_element_type=jnp.float32)
        for c in range(3):
            q = q - x1t[:, c:c + 1] * w1t_ref[c:c + 1, :]
        q_ref[...] = (q + b1_ref[...]).reshape(1, NN, CC)

    x2t = x2t_ref[...].reshape(NN, 3)
    x1c = xyz1_ref[0, :, pl.ds(t * TQ, TQ)]
    xx = (x2t[:, 0:1] * x2t[:, 0:1] + x2t[:, 1:2] * x2t[:, 1:2]
          + x2t[:, 2:3] * x2t[:, 2:3])
    yy = (x1c[0:1, :] * x1c[0:1, :] + x1c[1:2, :] * x1c[1:2, :]
          + x1c[2:3, :] * x1c[2:3, :])
    xy = jnp.dot(x2t, x1c, preferred_element_type=jnp.float32)
    d = (xx + yy) - 2.0 * xy
    dscr[...] = jnp.maximum(d, 0.0)

    iota_r = lax.broadcasted_iota(jnp.int32, (NN, TQ), 0)
    base = b * NN

    def tmin(v):
        size = v.shape[0] // 2
        while size >= 8:
            v = jnp.minimum(v[:size], v[size:])
            size //= 2
        return jnp.min(v, axis=0, keepdims=True)

    def sel(s, _):
        dd = dscr[...]
        m = tmin(dd)
        cand = jnp.where(dd == m, iota_r, jnp.int32(2 ** 30))
        jmin = tmin(cand)
        ind_ref[0, pl.ds(s, 1), :] = jmin + base
        dscr[...] = jnp.where(iota_r == jmin, BIG, dd)
        return 0

    lax.fori_loop(0, SS, sel, 0, unroll=4)

    @pl.when(jnp.logical_and(b == 0, t == 0))
    def _():
        stat_ref[...] = jnp.zeros((8, CC), jnp.float32)

    taken = jnp.where(dscr[...] == BIG, 1.0, 0.0)
    r_mat = lax.dot_general(taken, tabf_ref[...], (((0,), (0,)), ((), ())),
                            precision=_HIGH,
                            preferred_element_type=jnp.float32)
    cnt = jnp.sum(taken, axis=1, keepdims=True)
    t2 = lax.dot_general(cnt, tabsq_ref[...], (((0,), (0,)), ((), ())),
                         precision=_HIGH,
                         preferred_element_type=jnp.float32)
    q_tile = q_ref[0, pl.ds(t * TQ, TQ), :]
    s1 = (jnp.sum(r_mat, axis=0, keepdims=True)
          + float(SS) * jnp.sum(q_tile, axis=0, keepdims=True))
    s2 = (t2 + 2.0 * jnp.sum(q_tile * r_mat, axis=0, keepdims=True)
          + float(SS) * jnp.sum(q_tile * q_tile, axis=0, keepdims=True))
    stat_ref[...] += jnp.concatenate(
        [s1, s2, jnp.zeros((6, CC), jnp.float32)], axis=0)


def _k1(xyz1, x1t, x2t, f1t, f2t, w1t, b1):
    nt = NN // TQ
    return pl.pallas_call(
        _k1_body,
        grid=(HB, nt),
        in_specs=[
            pl.BlockSpec((1, 3, NN), lambda b, t: (b, 0, 0)),
            pl.BlockSpec((1, NN, 3), lambda b, t: (b, 0, 0)),
            pl.BlockSpec((1, NN, 3), lambda b, t: (b, 0, 0)),
            pl.BlockSpec((1, NN, CC), lambda b, t: (b, 0, 0)),
            pl.BlockSpec((1, NN, CC), lambda b, t: (b, 0, 0)),
            pl.BlockSpec((CIN, CC), lambda b, t: (0, 0)),
            pl.BlockSpec((1, CC), lambda b, t: (0, 0)),
        ],
        out_specs=[
            pl.BlockSpec((1, SS, TQ), lambda b, t: (b, 0, t)),
            pl.BlockSpec((1, NN, 2 * CC), lambda b, t: (b, 0, 0)),
            pl.BlockSpec((1, NN, CC), lambda b, t: (b, 0, 0)),
            pl.BlockSpec((8, CC), lambda b, t: (0, 0)),
        ],
        out_shape=[
            jax.ShapeDtypeStruct((HB, SS, NN), jnp.int32),
            jax.ShapeDtypeStruct((HB, NN, 2 * CC), jnp.float32),
            jax.ShapeDtypeStruct((HB, NN, CC), jnp.float32),
            jax.ShapeDtypeStruct((8, CC), jnp.float32),
        ],
        scratch_shapes=[pltpu.VMEM((NN, TQ), jnp.float32),
                        pltpu.VMEM((NN, CC), jnp.float32),
                        pltpu.VMEM((NN, CC), jnp.float32)],
    )(xyz1, x1t, x2t, f1t, f2t, w1t, b1)


_ROWS = HB * SS * NN
_CHUNK = 128


def _sc_gather(tab_flat, idx_flat):
    info = plsc.get_sparse_core_info()
    nw = info.num_cores * info.num_subcores
    cpw = _ROWS // (nw * _CHUNK)
    mesh = plsc.VectorSubcoreMesh(core_axis_name="c", subcore_axis_name="s")

    @functools.partial(
        pl.kernel,
        out_type=jax.ShapeDtypeStruct((_ROWS, 2 * CC), jnp.float32),
        mesh=mesh,
        scratch_types=[
            pltpu.VMEM((_CHUNK,), jnp.int32),
            pltpu.VMEM((_CHUNK, 2 * CC), jnp.float32),
            pltpu.SemaphoreType.DMA,
        ],
    )
    def k(tab_hbm, idx_hbm, out_hbm, idx_v, rows_v, sem):
        wid = lax.axis_index("s") * info.num_cores + lax.axis_index("c")

        def body(i, carry):
            base = (wid * cpw + i) * _CHUNK
            pltpu.sync_copy(idx_hbm.at[pl.ds(base, _CHUNK)], idx_v)
            pltpu.async_copy(tab_hbm.at[idx_v], rows_v, sem).wait()
            pltpu.sync_copy(rows_v, out_hbm.at[pl.ds(base, _CHUNK)])
            return carry

        lax.fori_loop(0, cpw, body, 0)

    return k(tab_flat, idx_flat)


def _stats_rows(y):
    s1 = jnp.sum(y, axis=0, keepdims=True)
    s2 = jnp.sum(y * y, axis=0, keepdims=True)
    return jnp.concatenate([s1, s2, jnp.zeros((6, CC), jnp.float32)], axis=0)


def _k2_body(y_ref, q_ref, a_ref, c_ref, w_ref, bias_ref, y2_ref, stat_ref):
    b, s = pl.program_id(0), pl.program_id(1)
    y = y_ref[...].reshape(NN, 2 * CC)[:, :CC] + q_ref[...].reshape(NN, CC)
    x = jax.nn.relu(y * a_ref[...] + c_ref[...])
    y2 = jnp.dot(x, w_ref[...], precision=_HIGH,
                 preferred_element_type=jnp.float32) + bias_ref[...]
    y2_ref[...] = y2.reshape(1, 1, NN, CC)

    @pl.when(jnp.logical_and(b == 0, s == 0))
    def _():
        stat_ref[...] = jnp.zeros((8, CC), jnp.float32)

    stat_ref[...] += _stats_rows(y2)


def _k2(y1_raw, q, a1, c1, w2t, b2):
    return pl.pallas_call(
        _k2_body,
        grid=(HB, SS),
        in_specs=[
            pl.BlockSpec((1, 1, NN, 2 * CC), lambda b, s: (b, s, 0, 0)),
            pl.BlockSpec((1, NN, CC), lambda b, s: (b, 0, 0)),
            pl.BlockSpec((1, CC), lambda b, s: (0, 0)),
            pl.BlockSpec((1, CC), lambda b, s: (0, 0)),
            pl.BlockSpec((CC, CC), lambda b, s: (0, 0)),
            pl.BlockSpec((1, CC), lambda b, s: (0, 0)),
        ],
        out_specs=[
            pl.BlockSpec((1, 1, NN, CC), lambda b, s: (b, s, 0, 0)),
            pl.BlockSpec((8, CC), lambda b, s: (0, 0)),
        ],
        out_shape=[
            jax.ShapeDtypeStruct((HB, SS, NN, CC), jnp.float32),
            jax.ShapeDtypeStruct((8, CC), jnp.float32),
        ],
    )(y1_raw, q, a1, c1, w2t, b2)


def _k3_body(y_ref, a_ref, c_ref, w_ref, bias_ref, y3_ref, stat_ref):
    b, s = pl.program_id(0), pl.program_id(1)
    x = jax.nn.relu(y_ref[...].reshape(NN, CC) * a_ref[...] + c_ref[...])
    y3 = jnp.dot(x, w_ref[...], precision=_HIGH,
                 preferred_element_type=jnp.float32) + bias_ref[...]
    y3_ref[...] = y3.reshape(1, 1, NN, CC)

    @pl.when(jnp.logical_and(b == 0, s == 0))
    def _():
        stat_ref[...] = jnp.zeros((8, CC), jnp.float32)

    stat_ref[...] += _stats_rows(y3)


def _k3(y2, a2, c2, w3t, b3):
    return pl.pallas_call(
        _k3_body,
        grid=(HB, SS),
        in_specs=[
            pl.BlockSpec((1, 1, NN, CC), lambda b, s: (b, s, 0, 0)),
            pl.BlockSpec((1, CC), lambda b, s: (0, 0)),
            pl.BlockSpec((1, CC), lambda b, s: (0, 0)),
            pl.BlockSpec((CC, CC), lambda b, s: (0, 0)),
            pl.BlockSpec((1, CC), lambda b, s: (0, 0)),
        ],
        out_specs=[
            pl.BlockSpec((1, 1, NN, CC), lambda b, s: (b, s, 0, 0)),
            pl.BlockSpec((8, CC), lambda b, s: (0, 0)),
        ],
        out_shape=[
            jax.ShapeDtypeStruct((HB, SS, NN, CC), jnp.float32),
            jax.ShapeDtypeStruct((8, CC), jnp.float32),
        ],
    )(y2, a2, c2, w3t, b3)


def _k4_body(y_ref, a_ref, c_ref, out_ref):
    s = pl.program_id(1)
    x = jax.nn.relu(y_ref[...].reshape(NN, CC) * a_ref[...] + c_ref[...])

    @pl.when(s == 0)
    def _():
        out_ref[...] = x.reshape(1, NN, CC)

    @pl.when(s != 0)
    def _():
        out_ref[...] = jnp.maximum(out_ref[...], x.reshape(1, NN, CC))


def _k4(y3, a3, c3):
    return pl.pallas_call(
        _k4_body,
        grid=(HB, SS),
        in_specs=[
            pl.BlockSpec((1, 1, NN, CC), lambda b, s: (b, s, 0, 0)),
            pl.BlockSpec((1, CC), lambda b, s: (0, 0)),
            pl.BlockSpec((1, CC), lambda b, s: (0, 0)),
        ],
        out_specs=pl.BlockSpec((1, NN, CC), lambda b, s: (b, 0, 0)),
        out_shape=jax.ShapeDtypeStruct((HB, NN, CC), jnp.float32),
    )(y3, a3, c3)


def _bn_coeffs(stat, g, be):
    cnt = jnp.float32(BB * SS * NN)
    mean = stat[0] / cnt
    var = stat[1] / cnt - mean * mean
    a = g / jnp.sqrt(var + EPS)
    c = be - mean * a
    return a.reshape(1, CC), c.reshape(1, CC)


def kernel(xyz1, xyz2, feat1, feat2,
           W1, b1, g1, be1, W2, b2, g2, be2, W3, b3, g3, be3):
    x1t = jnp.transpose(xyz1, (0, 2, 1))
    x2t = jnp.transpose(xyz2, (0, 2, 1))
    f1t = jnp.transpose(feat1, (0, 2, 1))
    f2t = jnp.transpose(feat2, (0, 2, 1))
    w1t = W1.T
    b1r = b1.reshape(1, CC)

    halves = []
    for h in range(NH):
        sl = slice(h * HB, (h + 1) * HB)
        ind, tab, q, st = _k1(xyz1[sl], x1t[sl], x2t[sl], f1t[sl], f2t[sl],
                              w1t, b1r)
        y1 = _sc_gather(tab.reshape(HB * NN, 2 * CC), ind.reshape(_ROWS))
        halves.append((y1.reshape(HB, SS, NN, 2 * CC), q, st))

    stat1 = halves[0][2] + halves[1][2]
    a1, c1 = _bn_coeffs(stat1, g1, be1)

    y2s = []
    for y1_raw, q, _ in halves:
        y2s.append(_k2(y1_raw, q, a1, c1, W2.T, b2.reshape(1, CC)))
    stat2 = y2s[0][1] + y2s[1][1]
    a2, c2 = _bn_coeffs(stat2, g2, be2)

    y3s = []
    for y2, _ in y2s:
        y3s.append(_k3(y2, a2, c2, W3.T, b3.reshape(1, CC)))
    stat3 = y3s[0][1] + y3s[1][1]
    a3, c3 = _bn_coeffs(stat3, g3, be3)

    outs = [_k4(y3, a3, c3) for y3, _ in y3s]
    out = jnp.concatenate(outs, axis=0)
    return jnp.transpose(out, (0, 2, 1))


# speedup vs baseline: 1.4301x; 1.4301x over previous
import functools

import jax
import jax.numpy as jnp
from jax import lax
from jax.experimental import pallas as pl
from jax.experimental.pallas import tpu as pltpu
from jax.experimental.pallas import tpu_sc as plsc

BB, NN, SS, CC = 4, 1024, 64, 64
HB = 2
NH = BB // HB
CIN = 2 * CC + 3
TQ = 1024
EPS = 1e-3
BIG = 3.0e38

_HIGH = lax.Precision.HIGHEST


def _k1_body(xyz1_ref, x1t_ref, x2t_ref, f1t_ref, f2t_ref, w1t_ref, b1_ref,
             ind_ref, tab_ref, q_ref, stat_ref, dscr, tabf_ref, tabsq_ref):
    b = pl.program_id(0)
    t = pl.program_id(1)

    @pl.when(t == 0)
    def _tables():
        x2t = x2t_ref[...].reshape(NN, 3)
        f2t = f2t_ref[...].reshape(NN, CC)
        acc = jnp.dot(f2t, w1t_ref[3:3 + CC, :], precision=_HIGH,
                      preferred_element_type=jnp.float32)
        for c in range(3):
            acc = acc + x2t[:, c:c + 1] * w1t_ref[c:c + 1, :]
        tab_ref[...] = jnp.concatenate(
            [acc, jnp.zeros((NN, CC), jnp.float32)], axis=1
        ).reshape(1, NN, 2 * CC)
        tabf_ref[...] = acc
        tabsq_ref[...] = acc * acc

        x1t = x1t_ref[...].reshape(NN, 3)
        f1t = f1t_ref[...].reshape(NN, CC)
        q = jnp.dot(f1t, w1t_ref[3 + CC:CIN, :], precision=_HIGH,
                    preferred_element_type=jnp.float32)
        for c in range(3):
            q = q - x1t[:, c:c + 1] * w1t_ref[c:c + 1, :]
        q_ref[...] = (q + b1_ref[...]).reshape(1, NN, CC)

    x2t = x2t_ref[...].reshape(NN, 3)
    x1c = xyz1_ref[0, :, pl.ds(t * TQ, TQ)]
    xx = (x2t[:, 0:1] * x2t[:, 0:1] + x2t[:, 1:2] * x2t[:, 1:2]
          + x2t[:, 2:3] * x2t[:, 2:3])
    yy = (x1c[0:1, :] * x1c[0:1, :] + x1c[1:2, :] * x1c[1:2, :]
          + x1c[2:3, :] * x1c[2:3, :])
    xy = jnp.dot(x2t, x1c, preferred_element_type=jnp.float32)
    d = (xx + yy) - 2.0 * xy
    dscr[...] = jnp.maximum(d, 0.0)

    iota_r = lax.broadcasted_iota(jnp.int32, (NN, TQ), 0)
    base = b * NN

    def sel(s, _):
        dd = dscr[...]
        m = jnp.min(dd, axis=0, keepdims=True)
        cand = jnp.where(dd == m, iota_r, jnp.int32(2 ** 30))
        jmin = jnp.min(cand, axis=0, keepdims=True)
        ind_ref[0, pl.ds(s, 1), :] = jmin + base
        dscr[...] = jnp.where(iota_r == jmin, BIG, dd)
        return 0

    lax.fori_loop(0, SS, sel, 0, unroll=4)

    @pl.when(jnp.logical_and(b == 0, t == 0))
    def _():
        stat_ref[...] = jnp.zeros((8, CC), jnp.float32)

    taken = jnp.where(dscr[...] == BIG, 1.0, 0.0)
    r_mat = lax.dot_general(taken, tabf_ref[...], (((0,), (0,)), ((), ())),
                            precision=_HIGH,
                            preferred_element_type=jnp.float32)
    cnt = jnp.sum(taken, axis=1, keepdims=True)
    t2 = lax.dot_general(cnt, tabsq_ref[...], (((0,), (0,)), ((), ())),
                         precision=_HIGH,
                         preferred_element_type=jnp.float32)
    q_tile = q_ref[0, pl.ds(t * TQ, TQ), :]
    s1 = (jnp.sum(r_mat, axis=0, keepdims=True)
          + float(SS) * jnp.sum(q_tile, axis=0, keepdims=True))
    s2 = (t2 + 2.0 * jnp.sum(q_tile * r_mat, axis=0, keepdims=True)
          + float(SS) * jnp.sum(q_tile * q_tile, axis=0, keepdims=True))
    stat_ref[...] += jnp.concatenate(
        [s1, s2, jnp.zeros((6, CC), jnp.float32)], axis=0)


def _k1(xyz1, x1t, x2t, f1t, f2t, w1t, b1):
    nt = NN // TQ
    return pl.pallas_call(
        _k1_body,
        grid=(HB, nt),
        in_specs=[
            pl.BlockSpec((1, 3, NN), lambda b, t: (b, 0, 0)),
            pl.BlockSpec((1, NN, 3), lambda b, t: (b, 0, 0)),
            pl.BlockSpec((1, NN, 3), lambda b, t: (b, 0, 0)),
            pl.BlockSpec((1, NN, CC), lambda b, t: (b, 0, 0)),
            pl.BlockSpec((1, NN, CC), lambda b, t: (b, 0, 0)),
            pl.BlockSpec((CIN, CC), lambda b, t: (0, 0)),
            pl.BlockSpec((1, CC), lambda b, t: (0, 0)),
        ],
        out_specs=[
            pl.BlockSpec((1, SS, TQ), lambda b, t: (b, 0, t)),
            pl.BlockSpec((1, NN, 2 * CC), lambda b, t: (b, 0, 0)),
            pl.BlockSpec((1, NN, CC), lambda b, t: (b, 0, 0)),
            pl.BlockSpec((8, CC), lambda b, t: (0, 0)),
        ],
        out_shape=[
            jax.ShapeDtypeStruct((HB, SS, NN), jnp.int32),
            jax.ShapeDtypeStruct((HB, NN, 2 * CC), jnp.float32),
            jax.ShapeDtypeStruct((HB, NN, CC), jnp.float32),
            jax.ShapeDtypeStruct((8, CC), jnp.float32),
        ],
        scratch_shapes=[pltpu.VMEM((NN, TQ), jnp.float32),
                        pltpu.VMEM((NN, CC), jnp.float32),
                        pltpu.VMEM((NN, CC), jnp.float32)],
    )(xyz1, x1t, x2t, f1t, f2t, w1t, b1)


_ROWS = HB * SS * NN
_CHUNK = 128


def _sc_gather(tab_flat, idx_flat):
    info = plsc.get_sparse_core_info()
    nw = info.num_cores * info.num_subcores
    cpw = _ROWS // (nw * _CHUNK)
    mesh = plsc.VectorSubcoreMesh(core_axis_name="c", subcore_axis_name="s")

    @functools.partial(
        pl.kernel,
        out_type=jax.ShapeDtypeStruct((_ROWS, 2 * CC), jnp.float32),
        mesh=mesh,
        scratch_types=[
            pltpu.VMEM((_CHUNK,), jnp.int32),
            pltpu.VMEM((_CHUNK, 2 * CC), jnp.float32),
            pltpu.SemaphoreType.DMA,
        ],
    )
    def k(tab_hbm, idx_hbm, out_hbm, idx_v, rows_v, sem):
        wid = lax.axis_index("s") * info.num_cores + lax.axis_index("c")

        def body(i, carry):
            base = (wid * cpw + i) * _CHUNK
            pltpu.sync_copy(idx_hbm.at[pl.ds(base, _CHUNK)], idx_v)
            pltpu.async_copy(tab_hbm.at[idx_v], rows_v, sem).wait()
            pltpu.sync_copy(rows_v, out_hbm.at[pl.ds(base, _CHUNK)])
            return carry

        lax.fori_loop(0, cpw, body, 0)

    return k(tab_flat, idx_flat)


def _stats_rows(y):
    s1 = jnp.sum(y, axis=0, keepdims=True)
    s2 = jnp.sum(y * y, axis=0, keepdims=True)
    return jnp.concatenate([s1, s2, jnp.zeros((6, CC), jnp.float32)], axis=0)


SB = 4


def _k2_body(y_ref, q_ref, a_ref, c_ref, w_ref, bias_ref, y2_ref, stat_ref):
    b, s = pl.program_id(0), pl.program_id(1)
    y = (y_ref[...].reshape(SB, NN, 2 * CC)[:, :, :CC]
         + q_ref[...].reshape(1, NN, CC))
    x = jax.nn.relu(y * a_ref[...] + c_ref[...]).reshape(SB * NN, CC)
    y2 = jnp.dot(x, w_ref[...], precision=_HIGH,
                 preferred_element_type=jnp.float32) + bias_ref[...]
    y2_ref[...] = y2.reshape(1, SB, NN, CC)

    @pl.when(jnp.logical_and(b == 0, s == 0))
    def _():
        stat_ref[...] = jnp.zeros((8, CC), jnp.float32)

    stat_ref[...] += _stats_rows(y2)


def _k2(y1_raw, q, a1, c1, w2t, b2):
    return pl.pallas_call(
        _k2_body,
        grid=(HB, SS // SB),
        in_specs=[
            pl.BlockSpec((1, SB, NN, 2 * CC), lambda b, s: (b, s, 0, 0)),
            pl.BlockSpec((1, NN, CC), lambda b, s: (b, 0, 0)),
            pl.BlockSpec((1, CC), lambda b, s: (0, 0)),
            pl.BlockSpec((1, CC), lambda b, s: (0, 0)),
            pl.BlockSpec((CC, CC), lambda b, s: (0, 0)),
            pl.BlockSpec((1, CC), lambda b, s: (0, 0)),
        ],
        out_specs=[
            pl.BlockSpec((1, SB, NN, CC), lambda b, s: (b, s, 0, 0)),
            pl.BlockSpec((8, CC), lambda b, s: (0, 0)),
        ],
        out_shape=[
            jax.ShapeDtypeStruct((HB, SS, NN, CC), jnp.float32),
            jax.ShapeDtypeStruct((8, CC), jnp.float32),
        ],
    )(y1_raw, q, a1, c1, w2t, b2)


def _k3_body(y_ref, a_ref, c_ref, w_ref, bias_ref, y3_ref, stat_ref):
    b, s = pl.program_id(0), pl.program_id(1)
    x = jax.nn.relu(y_ref[...].reshape(SB * NN, CC) * a_ref[...]
                    + c_ref[...])
    y3 = jnp.dot(x, w_ref[...], precision=_HIGH,
                 preferred_element_type=jnp.float32) + bias_ref[...]
    y3_ref[...] = y3.reshape(1, SB, NN, CC)

    @pl.when(jnp.logical_and(b == 0, s == 0))
    def _():
        stat_ref[...] = jnp.zeros((8, CC), jnp.float32)

    stat_ref[...] += _stats_rows(y3)


def _k3(y2, a2, c2, w3t, b3):
    return pl.pallas_call(
        _k3_body,
        grid=(HB, SS // SB),
        in_specs=[
            pl.BlockSpec((1, SB, NN, CC), lambda b, s: (b, s, 0, 0)),
            pl.BlockSpec((1, CC), lambda b, s: (0, 0)),
            pl.BlockSpec((1, CC), lambda b, s: (0, 0)),
            pl.BlockSpec((CC, CC), lambda b, s: (0, 0)),
            pl.BlockSpec((1, CC), lambda b, s: (0, 0)),
        ],
        out_specs=[
            pl.BlockSpec((1, SB, NN, CC), lambda b, s: (b, s, 0, 0)),
            pl.BlockSpec((8, CC), lambda b, s: (0, 0)),
        ],
        out_shape=[
            jax.ShapeDtypeStruct((HB, SS, NN, CC), jnp.float32),
            jax.ShapeDtypeStruct((8, CC), jnp.float32),
        ],
    )(y2, a2, c2, w3t, b3)


def _k4_body(y_ref, a_ref, c_ref, out_ref):
    s = pl.program_id(1)
    x = jax.nn.relu(y_ref[...].reshape(SB, NN, CC) * a_ref[...]
                    + c_ref[...])
    xm = jnp.max(x, axis=0).reshape(1, NN, CC)

    @pl.when(s == 0)
    def _():
        out_ref[...] = xm

    @pl.when(s != 0)
    def _():
        out_ref[...] = jnp.maximum(out_ref[...], xm)


def _k4(y3, a3, c3):
    return pl.pallas_call(
        _k4_body,
        grid=(HB, SS // SB),
        in_specs=[
            pl.BlockSpec((1, SB, NN, CC), lambda b, s: (b, s, 0, 0)),
            pl.BlockSpec((1, CC), lambda b, s: (0, 0)),
            pl.BlockSpec((1, CC), lambda b, s: (0, 0)),
        ],
        out_specs=pl.BlockSpec((1, NN, CC), lambda b, s: (b, 0, 0)),
        out_shape=jax.ShapeDtypeStruct((HB, NN, CC), jnp.float32),
    )(y3, a3, c3)


def _bn_coeffs(stat, g, be):
    cnt = jnp.float32(BB * SS * NN)
    mean = stat[0] / cnt
    var = stat[1] / cnt - mean * mean
    a = g / jnp.sqrt(var + EPS)
    c = be - mean * a
    return a.reshape(1, CC), c.reshape(1, CC)


def kernel(xyz1, xyz2, feat1, feat2,
           W1, b1, g1, be1, W2, b2, g2, be2, W3, b3, g3, be3):
    x1t = jnp.transpose(xyz1, (0, 2, 1))
    x2t = jnp.transpose(xyz2, (0, 2, 1))
    f1t = jnp.transpose(feat1, (0, 2, 1))
    f2t = jnp.transpose(feat2, (0, 2, 1))
    w1t = W1.T
    b1r = b1.reshape(1, CC)

    halves = []
    for h in range(NH):
        sl = slice(h * HB, (h + 1) * HB)
        ind, tab, q, st = _k1(xyz1[sl], x1t[sl], x2t[sl], f1t[sl], f2t[sl],
                              w1t, b1r)
        y1 = _sc_gather(tab.reshape(HB * NN, 2 * CC), ind.reshape(_ROWS))
        halves.append((y1.reshape(HB, SS, NN, 2 * CC), q, st))

    stat1 = halves[0][2] + halves[1][2]
    a1, c1 = _bn_coeffs(stat1, g1, be1)

    y2s = []
    for y1_raw, q, _ in halves:
        y2s.append(_k2(y1_raw, q, a1, c1, W2.T, b2.reshape(1, CC)))
    stat2 = y2s[0][1] + y2s[1][1]
    a2, c2 = _bn_coeffs(stat2, g2, be2)

    y3s = []
    for y2, _ in y2s:
        y3s.append(_k3(y2, a2, c2, W3.T, b3.reshape(1, CC)))
    stat3 = y3s[0][1] + y3s[1][1]
    a3, c3 = _bn_coeffs(stat3, g3, be3)

    outs = [_k4(y3, a3, c3) for y3, _ in y3s]
    out = jnp.concatenate(outs, axis=0)
    return jnp.transpose(out, (0, 2, 1))


# speedup vs baseline: 1.4765x; 1.0325x over previous
import functools

import jax
import jax.numpy as jnp
from jax import lax
from jax.experimental import pallas as pl
from jax.experimental.pallas import tpu as pltpu
from jax.experimental.pallas import tpu_sc as plsc

BB, NN, SS, CC = 4, 1024, 64, 64
HB = 2
NH = BB // HB
CIN = 2 * CC + 3
TQ = 1024
EPS = 1e-3
BIG = 3.0e38

_HIGH = lax.Precision.HIGHEST


def _k1_body(xyz1_ref, x1t_ref, x2t_ref, f1t_ref, f2t_ref, w1t_ref, b1_ref,
             ind_ref, tab_ref, q_ref, stat_ref, dscr, tabf_ref, tabsq_ref):
    b = pl.program_id(0)
    t = pl.program_id(1)

    @pl.when(t == 0)
    def _tables():
        x2t = x2t_ref[...].reshape(NN, 3)
        f2t = f2t_ref[...].reshape(NN, CC)
        acc = jnp.dot(f2t, w1t_ref[3:3 + CC, :], precision=_HIGH,
                      preferred_element_type=jnp.float32)
        for c in range(3):
            acc = acc + x2t[:, c:c + 1] * w1t_ref[c:c + 1, :]
        tab_ref[...] = jnp.concatenate(
            [acc, jnp.zeros((NN, CC), jnp.float32)], axis=1
        ).reshape(1, NN, 2 * CC)
        tabf_ref[...] = acc
        tabsq_ref[...] = acc * acc

        x1t = x1t_ref[...].reshape(NN, 3)
        f1t = f1t_ref[...].reshape(NN, CC)
        q = jnp.dot(f1t, w1t_ref[3 + CC:CIN, :], precision=_HIGH,
                    preferred_element_type=jnp.float32)
        for c in range(3):
            q = q - x1t[:, c:c + 1] * w1t_ref[c:c + 1, :]
        q_ref[...] = (q + b1_ref[...]).reshape(1, NN, CC)

    x2t = x2t_ref[...].reshape(NN, 3)
    x1c = xyz1_ref[0, :, pl.ds(t * TQ, TQ)]
    xx = (x2t[:, 0:1] * x2t[:, 0:1] + x2t[:, 1:2] * x2t[:, 1:2]
          + x2t[:, 2:3] * x2t[:, 2:3])
    yy = (x1c[0:1, :] * x1c[0:1, :] + x1c[1:2, :] * x1c[1:2, :]
          + x1c[2:3, :] * x1c[2:3, :])
    xy = jnp.dot(x2t, x1c, preferred_element_type=jnp.float32)
    d = (xx + yy) - 2.0 * xy
    dscr[...] = jnp.maximum(d, 0.0)

    iota_r = lax.broadcasted_iota(jnp.int32, (NN, TQ), 0)
    base = b * NN

    def sel(s, _):
        dd = dscr[...]
        m = jnp.min(dd, axis=0, keepdims=True)
        cand = jnp.where(dd == m, iota_r, jnp.int32(2 ** 30))
        jmin = jnp.min(cand, axis=0, keepdims=True)
        ind_ref[0, pl.ds(s, 1), :] = jmin + base
        dscr[...] = jnp.where(iota_r == jmin, BIG, dd)
        return 0

    lax.fori_loop(0, SS, sel, 0, unroll=4)

    @pl.when(jnp.logical_and(b == 0, t == 0))
    def _():
        stat_ref[...] = jnp.zeros((8, CC), jnp.float32)

    taken = jnp.where(dscr[...] == BIG, 1.0, 0.0)
    r_mat = lax.dot_general(taken, tabf_ref[...], (((0,), (0,)), ((), ())),
                            precision=_HIGH,
                            preferred_element_type=jnp.float32)
    cnt = jnp.sum(taken, axis=1, keepdims=True)
    t2 = lax.dot_general(cnt, tabsq_ref[...], (((0,), (0,)), ((), ())),
                         precision=_HIGH,
                         preferred_element_type=jnp.float32)
    q_tile = q_ref[0, pl.ds(t * TQ, TQ), :]
    s1 = (jnp.sum(r_mat, axis=0, keepdims=True)
          + float(SS) * jnp.sum(q_tile, axis=0, keepdims=True))
    s2 = (t2 + 2.0 * jnp.sum(q_tile * r_mat, axis=0, keepdims=True)
          + float(SS) * jnp.sum(q_tile * q_tile, axis=0, keepdims=True))
    stat_ref[...] += jnp.concatenate(
        [s1, s2, jnp.zeros((6, CC), jnp.float32)], axis=0)


def _k1(xyz1, x1t, x2t, f1t, f2t, w1t, b1):
    nt = NN // TQ
    return pl.pallas_call(
        _k1_body,
        grid=(HB, nt),
        in_specs=[
            pl.BlockSpec((1, 3, NN), lambda b, t: (b, 0, 0)),
            pl.BlockSpec((1, NN, 3), lambda b, t: (b, 0, 0)),
            pl.BlockSpec((1, NN, 3), lambda b, t: (b, 0, 0)),
            pl.BlockSpec((1, NN, CC), lambda b, t: (b, 0, 0)),
            pl.BlockSpec((1, NN, CC), lambda b, t: (b, 0, 0)),
            pl.BlockSpec((CIN, CC), lambda b, t: (0, 0)),
            pl.BlockSpec((1, CC), lambda b, t: (0, 0)),
        ],
        out_specs=[
            pl.BlockSpec((1, SS, TQ), lambda b, t: (b, 0, t)),
            pl.BlockSpec((1, NN, 2 * CC), lambda b, t: (b, 0, 0)),
            pl.BlockSpec((1, NN, CC), lambda b, t: (b, 0, 0)),
            pl.BlockSpec((8, CC), lambda b, t: (0, 0)),
        ],
        out_shape=[
            jax.ShapeDtypeStruct((HB, SS, NN), jnp.int32),
            jax.ShapeDtypeStruct((HB, NN, 2 * CC), jnp.float32),
            jax.ShapeDtypeStruct((HB, NN, CC), jnp.float32),
            jax.ShapeDtypeStruct((8, CC), jnp.float32),
        ],
        scratch_shapes=[pltpu.VMEM((NN, TQ), jnp.float32),
                        pltpu.VMEM((NN, CC), jnp.float32),
                        pltpu.VMEM((NN, CC), jnp.float32)],
    )(xyz1, x1t, x2t, f1t, f2t, w1t, b1)


_ROWS = HB * SS * NN
_CHUNK = 128


def _sc_gather(tab_flat, idx_flat):
    info = plsc.get_sparse_core_info()
    nw = info.num_cores * info.num_subcores
    cpw = _ROWS // (nw * _CHUNK)
    mesh = plsc.VectorSubcoreMesh(core_axis_name="c", subcore_axis_name="s")

    @functools.partial(
        pl.kernel,
        out_type=jax.ShapeDtypeStruct((_ROWS, 2 * CC), jnp.float32),
        mesh=mesh,
        scratch_types=[
            pltpu.VMEM((_CHUNK,), jnp.int32),
            pltpu.VMEM((_CHUNK, 2 * CC), jnp.float32),
            pltpu.SemaphoreType.DMA,
        ],
    )
    def k(tab_hbm, idx_hbm, out_hbm, idx_v, rows_v, sem):
        wid = lax.axis_index("s") * info.num_cores + lax.axis_index("c")

        def body(i, carry):
            base = (wid * cpw + i) * _CHUNK
            pltpu.sync_copy(idx_hbm.at[pl.ds(base, _CHUNK)], idx_v)
            pltpu.async_copy(tab_hbm.at[idx_v], rows_v, sem).wait()
            pltpu.sync_copy(rows_v, out_hbm.at[pl.ds(base, _CHUNK)])
            return carry

        lax.fori_loop(0, cpw, body, 0)

    return k(tab_flat, idx_flat)


def _stats_rows(y):
    s1 = jnp.sum(y, axis=0, keepdims=True)
    s2 = jnp.sum(y * y, axis=0, keepdims=True)
    return jnp.concatenate([s1, s2, jnp.zeros((6, CC), jnp.float32)], axis=0)


SB = 8


def _k2_body(y_ref, q_ref, a_ref, c_ref, w_ref, bias_ref, y2_ref, stat_ref):
    b, s = pl.program_id(0), pl.program_id(1)
    y = (y_ref[...].reshape(SB, NN, 2 * CC)[:, :, :CC]
         + q_ref[...].reshape(1, NN, CC))
    x = jax.nn.relu(y * a_ref[...] + c_ref[...]).reshape(SB * NN, CC)
    y2 = jnp.dot(x, w_ref[...], precision=_HIGH,
                 preferred_element_type=jnp.float32) + bias_ref[...]
    y2_ref[...] = y2.reshape(1, SB, NN, CC)

    @pl.when(jnp.logical_and(b == 0, s == 0))
    def _():
        stat_ref[...] = jnp.zeros((8, CC), jnp.float32)

    stat_ref[...] += _stats_rows(y2)


def _k2(y1_raw, q, a1, c1, w2t, b2):
    return pl.pallas_call(
        _k2_body,
        grid=(HB, SS // SB),
        in_specs=[
            pl.BlockSpec((1, SB, NN, 2 * CC), lambda b, s: (b, s, 0, 0)),
            pl.BlockSpec((1, NN, CC), lambda b, s: (b, 0, 0)),
            pl.BlockSpec((1, CC), lambda b, s: (0, 0)),
            pl.BlockSpec((1, CC), lambda b, s: (0, 0)),
            pl.BlockSpec((CC, CC), lambda b, s: (0, 0)),
            pl.BlockSpec((1, CC), lambda b, s: (0, 0)),
        ],
        out_specs=[
            pl.BlockSpec((1, SB, NN, CC), lambda b, s: (b, s, 0, 0)),
            pl.BlockSpec((8, CC), lambda b, s: (0, 0)),
        ],
        out_shape=[
            jax.ShapeDtypeStruct((HB, SS, NN, CC), jnp.float32),
            jax.ShapeDtypeStruct((8, CC), jnp.float32),
        ],
    )(y1_raw, q, a1, c1, w2t, b2)


def _k3_body(y_ref, a_ref, c_ref, w_ref, bias_ref, y3_ref, stat_ref):
    b, s = pl.program_id(0), pl.program_id(1)
    x = jax.nn.relu(y_ref[...].reshape(SB * NN, CC) * a_ref[...]
                    + c_ref[...])
    y3 = jnp.dot(x, w_ref[...], precision=_HIGH,
                 preferred_element_type=jnp.float32) + bias_ref[...]
    y3_ref[...] = y3.reshape(1, SB, NN, CC)

    @pl.when(jnp.logical_and(b == 0, s == 0))
    def _():
        stat_ref[...] = jnp.zeros((8, CC), jnp.float32)

    stat_ref[...] += _stats_rows(y3)


def _k3(y2, a2, c2, w3t, b3):
    return pl.pallas_call(
        _k3_body,
        grid=(HB, SS // SB),
        in_specs=[
            pl.BlockSpec((1, SB, NN, CC), lambda b, s: (b, s, 0, 0)),
            pl.BlockSpec((1, CC), lambda b, s: (0, 0)),
            pl.BlockSpec((1, CC), lambda b, s: (0, 0)),
            pl.BlockSpec((CC, CC), lambda b, s: (0, 0)),
            pl.BlockSpec((1, CC), lambda b, s: (0, 0)),
        ],
        out_specs=[
            pl.BlockSpec((1, SB, NN, CC), lambda b, s: (b, s, 0, 0)),
            pl.BlockSpec((8, CC), lambda b, s: (0, 0)),
        ],
        out_shape=[
            jax.ShapeDtypeStruct((HB, SS, NN, CC), jnp.float32),
            jax.ShapeDtypeStruct((8, CC), jnp.float32),
        ],
    )(y2, a2, c2, w3t, b3)


def _k4_body(y_ref, a_ref, c_ref, out_ref):
    s = pl.program_id(1)
    x = jax.nn.relu(y_ref[...].reshape(SB, NN, CC) * a_ref[...]
                    + c_ref[...])
    xm = jnp.max(x, axis=0).reshape(1, NN, CC)

    @pl.when(s == 0)
    def _():
        out_ref[...] = xm

    @pl.when(s != 0)
    def _():
        out_ref[...] = jnp.maximum(out_ref[...], xm)


def _k4(y3, a3, c3):
    return pl.pallas_call(
        _k4_body,
        grid=(HB, SS // SB),
        in_specs=[
            pl.BlockSpec((1, SB, NN, CC), lambda b, s: (b, s, 0, 0)),
            pl.BlockSpec((1, CC), lambda b, s: (0, 0)),
            pl.BlockSpec((1, CC), lambda b, s: (0, 0)),
        ],
        out_specs=pl.BlockSpec((1, NN, CC), lambda b, s: (b, 0, 0)),
        out_shape=jax.ShapeDtypeStruct((HB, NN, CC), jnp.float32),
    )(y3, a3, c3)


def _bn_coeffs(stat, g, be):
    cnt = jnp.float32(BB * SS * NN)
    mean = stat[0] / cnt
    var = stat[1] / cnt - mean * mean
    a = g / jnp.sqrt(var + EPS)
    c = be - mean * a
    return a.reshape(1, CC), c.reshape(1, CC)


def kernel(xyz1, xyz2, feat1, feat2,
           W1, b1, g1, be1, W2, b2, g2, be2, W3, b3, g3, be3):
    x1t = jnp.transpose(xyz1, (0, 2, 1))
    x2t = jnp.transpose(xyz2, (0, 2, 1))
    f1t = jnp.transpose(feat1, (0, 2, 1))
    f2t = jnp.transpose(feat2, (0, 2, 1))
    w1t = W1.T
    b1r = b1.reshape(1, CC)

    halves = []
    for h in range(NH):
        sl = slice(h * HB, (h + 1) * HB)
        ind, tab, q, st = _k1(xyz1[sl], x1t[sl], x2t[sl], f1t[sl], f2t[sl],
                              w1t, b1r)
        y1 = _sc_gather(tab.reshape(HB * NN, 2 * CC), ind.reshape(_ROWS))
        halves.append((y1.reshape(HB, SS, NN, 2 * CC), q, st))

    stat1 = halves[0][2] + halves[1][2]
    a1, c1 = _bn_coeffs(stat1, g1, be1)

    y2s = []
    for y1_raw, q, _ in halves:
        y2s.append(_k2(y1_raw, q, a1, c1, W2.T, b2.reshape(1, CC)))
    stat2 = y2s[0][1] + y2s[1][1]
    a2, c2 = _bn_coeffs(stat2, g2, be2)

    y3s = []
    for y2, _ in y2s:
        y3s.append(_k3(y2, a2, c2, W3.T, b3.reshape(1, CC)))
    stat3 = y3s[0][1] + y3s[1][1]
    a3, c3 = _bn_coeffs(stat3, g3, be3)

    outs = [_k4(y3, a3, c3) for y3, _ in y3s]
    out = jnp.concatenate(outs, axis=0)
    return jnp.transpose(out, (0, 2, 1))


# speedup vs baseline: 1.4833x; 1.0046x over previous
import functools

import jax
import jax.numpy as jnp
from jax import lax
from jax.experimental import pallas as pl
from jax.experimental.pallas import tpu as pltpu
from jax.experimental.pallas import tpu_sc as plsc

BB, NN, SS, CC = 4, 1024, 64, 64
HB = 1
NH = BB // HB
CIN = 2 * CC + 3
TQ = 1024
EPS = 1e-3
BIG = 3.0e38

_HIGH = lax.Precision.HIGHEST


def _k1_body(xyz1_ref, x1t_ref, x2t_ref, f1t_ref, f2t_ref, w1t_ref, b1_ref,
             ind_ref, tab_ref, q_ref, stat_ref, dscr, tabf_ref, tabsq_ref):
    b = pl.program_id(0)
    t = pl.program_id(1)

    @pl.when(t == 0)
    def _tables():
        x2t = x2t_ref[...].reshape(NN, 3)
        f2t = f2t_ref[...].reshape(NN, CC)
        acc = jnp.dot(f2t, w1t_ref[3:3 + CC, :], precision=_HIGH,
                      preferred_element_type=jnp.float32)
        for c in range(3):
            acc = acc + x2t[:, c:c + 1] * w1t_ref[c:c + 1, :]
        tab_ref[...] = jnp.concatenate(
            [acc, jnp.zeros((NN, CC), jnp.float32)], axis=1
        ).reshape(1, NN, 2 * CC)
        tabf_ref[...] = acc
        tabsq_ref[...] = acc * acc

        x1t = x1t_ref[...].reshape(NN, 3)
        f1t = f1t_ref[...].reshape(NN, CC)
        q = jnp.dot(f1t, w1t_ref[3 + CC:CIN, :], precision=_HIGH,
                    preferred_element_type=jnp.float32)
        for c in range(3):
            q = q - x1t[:, c:c + 1] * w1t_ref[c:c + 1, :]
        q_ref[...] = (q + b1_ref[...]).reshape(1, NN, CC)

    x2t = x2t_ref[...].reshape(NN, 3)
    x1c = xyz1_ref[0, :, pl.ds(t * TQ, TQ)]
    xx = (x2t[:, 0:1] * x2t[:, 0:1] + x2t[:, 1:2] * x2t[:, 1:2]
          + x2t[:, 2:3] * x2t[:, 2:3])
    yy = (x1c[0:1, :] * x1c[0:1, :] + x1c[1:2, :] * x1c[1:2, :]
          + x1c[2:3, :] * x1c[2:3, :])
    xy = jnp.dot(x2t, x1c, preferred_element_type=jnp.float32)
    d = (xx + yy) - 2.0 * xy
    dscr[...] = jnp.maximum(d, 0.0)

    iota_r = lax.broadcasted_iota(jnp.int32, (NN, TQ), 0)
    base = b * NN

    def sel(s, _):
        dd = dscr[...]
        m = jnp.min(dd, axis=0, keepdims=True)
        cand = jnp.where(dd == m, iota_r, jnp.int32(2 ** 30))
        jmin = jnp.min(cand, axis=0, keepdims=True)
        ind_ref[0, pl.ds(s, 1), :] = jmin + base
        dscr[...] = jnp.where(iota_r == jmin, BIG, dd)
        return 0

    lax.fori_loop(0, SS, sel, 0, unroll=4)

    @pl.when(jnp.logical_and(b == 0, t == 0))
    def _():
        stat_ref[...] = jnp.zeros((8, CC), jnp.float32)

    taken = jnp.where(dscr[...] == BIG, 1.0, 0.0)
    r_mat = lax.dot_general(taken, tabf_ref[...], (((0,), (0,)), ((), ())),
                            precision=_HIGH,
                            preferred_element_type=jnp.float32)
    cnt = jnp.sum(taken, axis=1, keepdims=True)
    t2 = lax.dot_general(cnt, tabsq_ref[...], (((0,), (0,)), ((), ())),
                         precision=_HIGH,
                         preferred_element_type=jnp.float32)
    q_tile = q_ref[0, pl.ds(t * TQ, TQ), :]
    s1 = (jnp.sum(r_mat, axis=0, keepdims=True)
          + float(SS) * jnp.sum(q_tile, axis=0, keepdims=True))
    s2 = (t2 + 2.0 * jnp.sum(q_tile * r_mat, axis=0, keepdims=True)
          + float(SS) * jnp.sum(q_tile * q_tile, axis=0, keepdims=True))
    stat_ref[...] += jnp.concatenate(
        [s1, s2, jnp.zeros((6, CC), jnp.float32)], axis=0)


def _k1(xyz1, x1t, x2t, f1t, f2t, w1t, b1):
    nt = NN // TQ
    return pl.pallas_call(
        _k1_body,
        grid=(HB, nt),
        in_specs=[
            pl.BlockSpec((1, 3, NN), lambda b, t: (b, 0, 0)),
            pl.BlockSpec((1, NN, 3), lambda b, t: (b, 0, 0)),
            pl.BlockSpec((1, NN, 3), lambda b, t: (b, 0, 0)),
            pl.BlockSpec((1, NN, CC), lambda b, t: (b, 0, 0)),
            pl.BlockSpec((1, NN, CC), lambda b, t: (b, 0, 0)),
            pl.BlockSpec((CIN, CC), lambda b, t: (0, 0)),
            pl.BlockSpec((1, CC), lambda b, t: (0, 0)),
        ],
        out_specs=[
            pl.BlockSpec((1, SS, TQ), lambda b, t: (b, 0, t)),
            pl.BlockSpec((1, NN, 2 * CC), lambda b, t: (b, 0, 0)),
            pl.BlockSpec((1, NN, CC), lambda b, t: (b, 0, 0)),
            pl.BlockSpec((8, CC), lambda b, t: (0, 0)),
        ],
        out_shape=[
            jax.ShapeDtypeStruct((HB, SS, NN), jnp.int32),
            jax.ShapeDtypeStruct((HB, NN, 2 * CC), jnp.float32),
            jax.ShapeDtypeStruct((HB, NN, CC), jnp.float32),
            jax.ShapeDtypeStruct((8, CC), jnp.float32),
        ],
        scratch_shapes=[pltpu.VMEM((NN, TQ), jnp.float32),
                        pltpu.VMEM((NN, CC), jnp.float32),
                        pltpu.VMEM((NN, CC), jnp.float32)],
    )(xyz1, x1t, x2t, f1t, f2t, w1t, b1)


_ROWS = HB * SS * NN
_CHUNK = 128


def _sc_gather(tab_flat, idx_flat):
    info = plsc.get_sparse_core_info()
    nw = info.num_cores * info.num_subcores
    cpw = _ROWS // (nw * _CHUNK)
    mesh = plsc.VectorSubcoreMesh(core_axis_name="c", subcore_axis_name="s")

    @functools.partial(
        pl.kernel,
        out_type=jax.ShapeDtypeStruct((_ROWS, 2 * CC), jnp.float32),
        mesh=mesh,
        scratch_types=[
            pltpu.VMEM((_CHUNK,), jnp.int32),
            pltpu.VMEM((_CHUNK, 2 * CC), jnp.float32),
            pltpu.SemaphoreType.DMA,
        ],
    )
    def k(tab_hbm, idx_hbm, out_hbm, idx_v, rows_v, sem):
        wid = lax.axis_index("s") * info.num_cores + lax.axis_index("c")

        def body(i, carry):
            base = (wid * cpw + i) * _CHUNK
            pltpu.sync_copy(idx_hbm.at[pl.ds(base, _CHUNK)], idx_v)
            pltpu.async_copy(tab_hbm.at[idx_v], rows_v, sem).wait()
            pltpu.sync_copy(rows_v, out_hbm.at[pl.ds(base, _CHUNK)])
            return carry

        lax.fori_loop(0, cpw, body, 0)

    return k(tab_flat, idx_flat)


def _stats_rows(y):
    s1 = jnp.sum(y, axis=0, keepdims=True)
    s2 = jnp.sum(y * y, axis=0, keepdims=True)
    return jnp.concatenate([s1, s2, jnp.zeros((6, CC), jnp.float32)], axis=0)


SB = 8


def _k2_body(y_ref, q_ref, a_ref, c_ref, w_ref, bias_ref, y2_ref, stat_ref):
    b, s = pl.program_id(0), pl.program_id(1)
    y = (y_ref[...].reshape(SB, NN, 2 * CC)[:, :, :CC]
         + q_ref[...].reshape(1, NN, CC))
    x = jax.nn.relu(y * a_ref[...] + c_ref[...]).reshape(SB * NN, CC)
    y2 = jnp.dot(x, w_ref[...], precision=_HIGH,
                 preferred_element_type=jnp.float32) + bias_ref[...]
    y2_ref[...] = y2.reshape(1, SB, NN, CC)

    @pl.when(jnp.logical_and(b == 0, s == 0))
    def _():
        stat_ref[...] = jnp.zeros((8, CC), jnp.float32)

    stat_ref[...] += _stats_rows(y2)


def _k2(y1_raw, q, a1, c1, w2t, b2):
    return pl.pallas_call(
        _k2_body,
        grid=(HB, SS // SB),
        in_specs=[
            pl.BlockSpec((1, SB, NN, 2 * CC), lambda b, s: (b, s, 0, 0)),
            pl.BlockSpec((1, NN, CC), lambda b, s: (b, 0, 0)),
            pl.BlockSpec((1, CC), lambda b, s: (0, 0)),
            pl.BlockSpec((1, CC), lambda b, s: (0, 0)),
            pl.BlockSpec((CC, CC), lambda b, s: (0, 0)),
            pl.BlockSpec((1, CC), lambda b, s: (0, 0)),
        ],
        out_specs=[
            pl.BlockSpec((1, SB, NN, CC), lambda b, s: (b, s, 0, 0)),
            pl.BlockSpec((8, CC), lambda b, s: (0, 0)),
        ],
        out_shape=[
            jax.ShapeDtypeStruct((HB, SS, NN, CC), jnp.float32),
            jax.ShapeDtypeStruct((8, CC), jnp.float32),
        ],
    )(y1_raw, q, a1, c1, w2t, b2)


def _k3_body(y_ref, a_ref, c_ref, w_ref, bias_ref, y3_ref, stat_ref):
    b, s = pl.program_id(0), pl.program_id(1)
    x = jax.nn.relu(y_ref[...].reshape(SB * NN, CC) * a_ref[...]
                    + c_ref[...])
    y3 = jnp.dot(x, w_ref[...], precision=_HIGH,
                 preferred_element_type=jnp.float32) + bias_ref[...]
    y3_ref[...] = y3.reshape(1, SB, NN, CC)

    @pl.when(jnp.logical_and(b == 0, s == 0))
    def _():
        stat_ref[...] = jnp.zeros((8, CC), jnp.float32)

    stat_ref[...] += _stats_rows(y3)


def _k3(y2, a2, c2, w3t, b3):
    return pl.pallas_call(
        _k3_body,
        grid=(HB, SS // SB),
        in_specs=[
            pl.BlockSpec((1, SB, NN, CC), lambda b, s: (b, s, 0, 0)),
            pl.BlockSpec((1, CC), lambda b, s: (0, 0)),
            pl.BlockSpec((1, CC), lambda b, s: (0, 0)),
            pl.BlockSpec((CC, CC), lambda b, s: (0, 0)),
            pl.BlockSpec((1, CC), lambda b, s: (0, 0)),
        ],
        out_specs=[
            pl.BlockSpec((1, SB, NN, CC), lambda b, s: (b, s, 0, 0)),
            pl.BlockSpec((8, CC), lambda b, s: (0, 0)),
        ],
        out_shape=[
            jax.ShapeDtypeStruct((HB, SS, NN, CC), jnp.float32),
            jax.ShapeDtypeStruct((8, CC), jnp.float32),
        ],
    )(y2, a2, c2, w3t, b3)


def _k4_body(y_ref, a_ref, c_ref, out_ref):
    s = pl.program_id(1)
    x = jax.nn.relu(y_ref[...].reshape(SB, NN, CC) * a_ref[...]
                    + c_ref[...])
    xm = jnp.max(x, axis=0).reshape(1, NN, CC)

    @pl.when(s == 0)
    def _():
        out_ref[...] = xm

    @pl.when(s != 0)
    def _():
        out_ref[...] = jnp.maximum(out_ref[...], xm)


def _k4(y3, a3, c3):
    return pl.pallas_call(
        _k4_body,
        grid=(HB, SS // SB),
        in_specs=[
            pl.BlockSpec((1, SB, NN, CC), lambda b, s: (b, s, 0, 0)),
            pl.BlockSpec((1, CC), lambda b, s: (0, 0)),
            pl.BlockSpec((1, CC), lambda b, s: (0, 0)),
        ],
        out_specs=pl.BlockSpec((1, NN, CC), lambda b, s: (b, 0, 0)),
        out_shape=jax.ShapeDtypeStruct((HB, NN, CC), jnp.float32),
    )(y3, a3, c3)


def _bn_coeffs(stat, g, be):
    cnt = jnp.float32(BB * SS * NN)
    mean = stat[0] / cnt
    var = stat[1] / cnt - mean * mean
    a = g / jnp.sqrt(var + EPS)
    c = be - mean * a
    return a.reshape(1, CC), c.reshape(1, CC)


def kernel(xyz1, xyz2, feat1, feat2,
           W1, b1, g1, be1, W2, b2, g2, be2, W3, b3, g3, be3):
    x1t = jnp.transpose(xyz1, (0, 2, 1))
    x2t = jnp.transpose(xyz2, (0, 2, 1))
    f1t = jnp.transpose(feat1, (0, 2, 1))
    f2t = jnp.transpose(feat2, (0, 2, 1))
    w1t = W1.T
    b1r = b1.reshape(1, CC)

    halves = []
    for h in range(NH):
        sl = slice(h * HB, (h + 1) * HB)
        ind, tab, q, st = _k1(xyz1[sl], x1t[sl], x2t[sl], f1t[sl], f2t[sl],
                              w1t, b1r)
        y1 = _sc_gather(tab.reshape(HB * NN, 2 * CC), ind.reshape(_ROWS))
        halves.append((y1.reshape(HB, SS, NN, 2 * CC), q, st))

    stat1 = sum(h[2] for h in halves)
    a1, c1 = _bn_coeffs(stat1, g1, be1)

    y2s = []
    for y1_raw, q, _ in halves:
        y2s.append(_k2(y1_raw, q, a1, c1, W2.T, b2.reshape(1, CC)))
    stat2 = sum(p[1] for p in y2s)
    a2, c2 = _bn_coeffs(stat2, g2, be2)

    y3s = []
    for y2, _ in y2s:
        y3s.append(_k3(y2, a2, c2, W3.T, b3.reshape(1, CC)))
    stat3 = sum(p[1] for p in y3s)
    a3, c3 = _bn_coeffs(stat3, g3, be3)

    outs = [_k4(y3, a3, c3) for y3, _ in y3s]
    out = jnp.concatenate(outs, axis=0)
    return jnp.transpose(out, (0, 2, 1))
